# baseline (speedup 1.0000x reference)
import functools

import jax
import jax.numpy as jnp
from jax import lax
from jax.experimental import pallas as pl
from jax.experimental.pallas import tpu as pltpu
from jax.experimental.pallas import tpu_sc as plsc

D_MODEL = 32
LANES = 128
NW = 32
NSUB = 8
NBUF = 2
KTILES = D_MODEL // 8


def _make_kernel(n_seq: int, n_batch: int, interpret: bool = False):
    tcs = n_batch // LANES
    n_units = n_seq * tcs
    units_per_w = n_units // NW
    n_chunks = units_per_w // NSUB
    assert tcs % NSUB == 0 and n_chunks > 1
    mesh = plsc.VectorSubcoreMesh(core_axis_name="c", subcore_axis_name="s", num_cores=2, num_subcores=16)

    @functools.partial(
        pl.kernel,
        out_type=jax.ShapeDtypeStruct(
            (n_seq, KTILES, tcs, 8, LANES), jnp.float32
        ),
        mesh=mesh,
        scratch_types=(
            [pltpu.VMEM((NSUB, LANES), jnp.int32) for _ in range(NBUF)]
            + [pltpu.VMEM((NSUB, LANES, D_MODEL), jnp.float32) for _ in range(NBUF)]
            + [pltpu.SemaphoreType.DMA for _ in range(2 * NBUF)]
        ),
        compiler_params=pltpu.CompilerParams(use_tc_tiling_on_sc=False),
        interpret=interpret,
    )
    def gather_kernel(idx_hbm, tab_hbm, out_hbm, *scratch):
        idxs = scratch[:NBUF]
        rows = scratch[NBUF:2 * NBUF]
        gsems = scratch[2 * NBUF:3 * NBUF]
        osems = scratch[3 * NBUF:]

        wid = lax.axis_index("s") * 2 + lax.axis_index("c")
        unit0 = wid * units_per_w

        def fire(g, b):
            u0 = unit0 + g * NSUB
            pltpu.sync_copy(idx_hbm.at[pl.ds(u0, NSUB)], idxs[b])
            for m in range(NSUB):
                pltpu.async_copy(tab_hbm.at[idxs[b].at[m]], rows[b].at[m], gsems[b])

        def wait_gathers(b):
            for m in range(NSUB):
                pltpu.make_async_copy(tab_hbm.at[pl.ds(0, LANES)],
                                      rows[b].at[m], gsems[b]).wait()

        def do_writes(g, b):
            u0 = unit0 + g * NSUB
            j = u0 // tcs
            tc0 = lax.rem(u0, tcs)
            copies = []
            for k in range(D_MODEL):
                tr, r = k // 8, k % 8
                copies.append(pltpu.async_copy(
                    rows[b].at[:, :, k],
                    out_hbm.at[j, tr, pl.ds(tc0, NSUB), r],
                    osems[b],
                ))
            for c in copies:
                c.wait()

        fire(0, 0)

        @pl.loop(0, n_chunks, step=NBUF)
        def body(t):
            for bb in range(NBUF):
                g = t + bb
                nb = 1 - bb

                @pl.when(g + 1 < n_chunks)
                def _():
                    fire(g + 1, nb)

                wait_gathers(bb)
                do_writes(g, bb)

    return gather_kernel


@jax.jit
def kernel(stem_idx, embedding_weight):
    bsz, seq = stem_idx.shape
    idx2d = jnp.swapaxes(stem_idx, 0, 1).astype(jnp.int32).reshape(-1, LANES)
    out5d = _make_kernel(seq, bsz)(idx2d, embedding_weight)
    out = out5d.transpose((2, 4, 0, 1, 3)).reshape(bsz, seq, D_MODEL)
    return out


# speedup vs baseline: 96.6280x; 96.6280x over previous
import functools

import jax
import jax.numpy as jnp
from jax import lax
from jax.experimental import pallas as pl
from jax.experimental.pallas import tpu as pltpu
from jax.experimental.pallas import tpu_sc as plsc

D_MODEL = 32
LANES = 128
NW = 32
NSUB = 4
NBUF = 4
LOOKAHEAD = 2


def _make_kernel(n_rows: int):
    rows_per_w = n_rows // NW
    n_chunks = rows_per_w // NSUB
    assert n_chunks % NBUF == 0 and n_chunks > NBUF
    mesh = plsc.VectorSubcoreMesh(core_axis_name="c", subcore_axis_name="s")

    @functools.partial(
        pl.kernel,
        out_type=jax.ShapeDtypeStruct((n_rows, LANES, D_MODEL), jnp.float32),
        mesh=mesh,
        scratch_types=(
            [pltpu.VMEM((NSUB, LANES), jnp.int32) for _ in range(NBUF)]
            + [pltpu.VMEM((NSUB, LANES, D_MODEL), jnp.float32) for _ in range(NBUF)]
            + [pltpu.SemaphoreType.DMA for _ in range(2 * NBUF)]
        ),
        compiler_params=pltpu.CompilerParams(use_tc_tiling_on_sc=False),
    )
    def gather_kernel(idx_hbm, tab_hbm, out_hbm, *scratch):
        idxs = scratch[:NBUF]
        rows = scratch[NBUF:2 * NBUF]
        gsems = scratch[2 * NBUF:3 * NBUF]
        osems = scratch[3 * NBUF:]

        wid = lax.axis_index("s") * 2 + lax.axis_index("c")
        row0 = wid * rows_per_w

        def fire(g, b):
            r0 = row0 + g * NSUB
            pltpu.sync_copy(idx_hbm.at[pl.ds(r0, NSUB)], idxs[b])
            for j in range(NSUB):
                pltpu.async_copy(tab_hbm.at[idxs[b].at[j]], rows[b].at[j], gsems[b])

        def wait_gathers(b):
            pltpu.make_async_copy(out_hbm.at[pl.ds(0, NSUB)], rows[b], gsems[b]).wait()

        def wait_writeout(b):
            pltpu.make_async_copy(out_hbm.at[pl.ds(0, NSUB)], rows[b], osems[b]).wait()

        for g in range(LOOKAHEAD):
            fire(g, g % NBUF)

        @pl.loop(0, n_chunks, step=NBUF)
        def body(t):
            for b in range(NBUF):
                g = t + b
                bn = (b + LOOKAHEAD) % NBUF
                nxt = g + LOOKAHEAD

                @pl.when(nxt < n_chunks)
                def _():
                    @pl.when(nxt >= NBUF)
                    def _():
                        wait_writeout(bn)
                    fire(nxt, bn)

                wait_gathers(b)
                pltpu.async_copy(
                    rows[b], out_hbm.at[pl.ds(row0 + g * NSUB, NSUB)], osems[b]
                )

        for b in range(NBUF):
            wait_writeout(b)

    return gather_kernel


@jax.jit
def kernel(stem_idx, embedding_weight):
    b, s = stem_idx.shape
    n = b * s
    idx2d = stem_idx.astype(jnp.int32).reshape(n // LANES, LANES)
    out = _make_kernel(n // LANES)(idx2d, embedding_weight)
    return out.reshape(b, s, D_MODEL)


# speedup vs baseline: 209.0087x; 2.1630x over previous
import functools

import jax
import jax.numpy as jnp
from jax import lax
from jax.experimental import pallas as pl
from jax.experimental.pallas import tpu as pltpu
from jax.experimental.pallas import tpu_sc as plsc

D_MODEL = 32
LANES = 128
NW = 32
NSUB = 4
NBUF = 4
LOOKAHEAD = 2
KTILES = D_MODEL // 8


def _make_kernel(n_rows: int):
    rows_per_w = n_rows // NW
    n_chunks = rows_per_w // NSUB
    assert n_chunks % NBUF == 0 and n_chunks > NBUF
    mesh = plsc.VectorSubcoreMesh(core_axis_name="c", subcore_axis_name="s")

    @functools.partial(
        pl.kernel,
        out_type=jax.ShapeDtypeStruct((n_rows, LANES, D_MODEL), jnp.float32),
        mesh=mesh,
        scratch_types=(
            [pltpu.VMEM((NSUB, LANES), jnp.int32) for _ in range(NBUF)]
            + [pltpu.VMEM((NSUB, LANES, D_MODEL), jnp.float32) for _ in range(NBUF)]
            + [pltpu.SemaphoreType.DMA for _ in range(2 * NBUF)]
        ),
        compiler_params=pltpu.CompilerParams(use_tc_tiling_on_sc=False),
    )
    def gather_kernel(idx_hbm, tab_hbm, out_hbm, *scratch):
        idxs = scratch[:NBUF]
        rows = scratch[NBUF:2 * NBUF]
        gsems = scratch[2 * NBUF:3 * NBUF]
        osems = scratch[3 * NBUF:]

        wid = lax.axis_index("s") * 2 + lax.axis_index("c")
        row0 = wid * rows_per_w

        def fire(g, b):
            r0 = row0 + g * NSUB
            pltpu.sync_copy(idx_hbm.at[pl.ds(r0, NSUB)], idxs[b])
            for j in range(NSUB):
                pltpu.async_copy(tab_hbm.at[idxs[b].at[j]], rows[b].at[j], gsems[b])

        def wait_gathers(b):
            pltpu.make_async_copy(out_hbm.at[pl.ds(0, NSUB)], rows[b], gsems[b]).wait()

        def wait_writeout(b):
            pltpu.make_async_copy(out_hbm.at[pl.ds(0, NSUB)], rows[b], osems[b]).wait()

        for g in range(LOOKAHEAD):
            fire(g, g % NBUF)

        @pl.loop(0, n_chunks, step=NBUF)
        def body(t):
            for b in range(NBUF):
                g = t + b
                bn = (b + LOOKAHEAD) % NBUF
                nxt = g + LOOKAHEAD

                @pl.when(nxt < n_chunks)
                def _():
                    @pl.when(nxt >= NBUF)
                    def _():
                        wait_writeout(bn)
                    fire(nxt, bn)

                wait_gathers(b)
                pltpu.async_copy(
                    rows[b], out_hbm.at[pl.ds(row0 + g * NSUB, NSUB)], osems[b]
                )

        for b in range(NBUF):
            wait_writeout(b)

    return gather_kernel


def _make_tc_transpose(n_batch: int, n_seq: int):
    tcs = n_batch // LANES
    jk = n_seq * D_MODEL
    blk_rows = LANES * jk // LANES

    def tc_body(in_ref, out_ref):
        x = in_ref[...]
        out_ref[...] = (
            x.reshape(LANES, jk).swapaxes(0, 1)
            .reshape(n_seq, KTILES, 1, 8, LANES)
        )

    return pl.pallas_call(
        tc_body,
        out_shape=jax.ShapeDtypeStruct(
            (n_seq, KTILES, tcs, 8, LANES), jnp.float32
        ),
        grid=(tcs,),
        in_specs=[pl.BlockSpec((blk_rows, LANES), lambda t: (t, 0))],
        out_specs=pl.BlockSpec(
            (n_seq, KTILES, 1, 8, LANES), lambda t: (0, 0, t, 0, 0)
        ),
    )


@jax.jit
def kernel(stem_idx, embedding_weight):
    b, s = stem_idx.shape
    n = b * s
    idx2d = stem_idx.astype(jnp.int32).reshape(n // LANES, LANES)
    out = _make_kernel(n // LANES)(idx2d, embedding_weight)
    out5d = _make_tc_transpose(b, s)(out.reshape(n * D_MODEL // LANES, LANES))
    return out5d.transpose((2, 4, 0, 1, 3)).reshape(b, s, D_MODEL)
